# baseline (speedup 1.0000x reference)
import functools

import jax
import jax.numpy as jnp
from jax import lax
from jax.experimental import pallas as pl
from jax.experimental.pallas import tpu as pltpu
from jax.experimental.pallas import tpu_sc as plsc

N = 10000
E = 320000
C = 128
G = 16
NW = 32
CHUNK = 128
NROWCHUNKS = 2560
EPAD = NROWCHUNKS * CHUNK - E
NACC = N + 8
GROUP = 8
WCHUNKS = NROWCHUNKS // NW
NGROUPS = WCHUNKS // GROUP
DEPTH = 2
FASTCID = 1
QONLY = NROWCHUNKS // 16
STRIPE = 624
LASTROWS = N - 15 * STRIPE
LASTACC = NACC - 15 * STRIPE

_HIGH = jax.lax.Precision.HIGHEST
IB = 16
NB = WCHUNKS // IB


def _leaky(v):
    return jnp.where(v > 0, v, 0.01 * v)


def _dot(a, b):
    return jnp.dot(a, b, precision=_HIGH, preferred_element_type=jnp.float32)


@functools.lru_cache(maxsize=None)
def _sc_scatter_fn():
    mesh = plsc.VectorSubcoreMesh(core_axis_name="c", subcore_axis_name="s")

    @functools.partial(
        pl.kernel,
        mesh=mesh,
        out_type=jax.ShapeDtypeStruct((N, C), jnp.float32),
        scratch_types=[
            pltpu.VMEM((IB, 2, CHUNK), jnp.int32),
            pltpu.VMEM((IB, 2, CHUNK), jnp.int32),
            *[pltpu.VMEM((CHUNK, C), jnp.float32) for _ in range(DEPTH)],
            pltpu.VMEM_SHARED((NACC, C), jnp.float32),
            pltpu.SemaphoreType.DMA,
            pltpu.SemaphoreType.DMA,
            pltpu.SemaphoreType.DMA,
            pltpu.SemaphoreType.DMA,
        ],
    )
    def k(x_hbm, sd_hbm, zeros_hbm, out_hbm,
          ib0, ib1, r0, r1, shared, isem, gsem0, gsem1, ssem):
        rows = [r0, r1]
        gsems = [gsem0, gsem1]
        ibufs = [ib0, ib1]
        cid = lax.axis_index("c")
        sid = lax.axis_index("s")

        @pl.when(cid == FASTCID)
        def _():
            pltpu.sync_copy(zeros_hbm.at[pl.ds(0, STRIPE)],
                            shared.at[pl.ds(sid * STRIPE, STRIPE)])

        @pl.when((sid == 15) & (cid == FASTCID))
        def _():
            pltpu.sync_copy(zeros_hbm.at[pl.ds(0, LASTACC - STRIPE)],
                            shared.at[pl.ds(16 * STRIPE,
                                            LASTACC - STRIPE)])

        plsc.subcore_barrier()

        def _ring(base, nch):
            nbk = nch // IB
            icps = [None] * nbk
            icps[0] = pltpu.async_copy(sd_hbm.at[pl.ds(base, IB)], ib0,
                                       isem)
            gcps = [None] * nch
            scps = [None] * nch

            def _scat(j):
                gcps[j].wait()
                jb = ibufs[(j // IB) % 2]
                scps[j] = pltpu.async_copy(
                    rows[j % DEPTH], shared.at[jb.at[j % IB, 1]], ssem,
                    add=True)

            for c in range(nch):
                blk, off = divmod(c, IB)
                if off == 0:
                    icps[blk].wait()
                if off == 2 and blk + 1 < nbk:
                    icps[blk + 1] = pltpu.async_copy(
                        sd_hbm.at[pl.ds(base + (blk + 1) * IB, IB)],
                        ibufs[(blk + 1) % 2], isem)
                if c >= DEPTH:
                    scps[c - DEPTH].wait()
                gcps[c] = pltpu.async_copy(
                    x_hbm.at[ibufs[blk % 2].at[off, 0]], rows[c % DEPTH],
                    gsems[c % 2])
                if c >= 1:
                    _scat(c - 1)
            _scat(nch - 1)
            scps[nch - 2].wait()
            scps[nch - 1].wait()

        @pl.when(cid == FASTCID)
        def _():
            _ring(sid * QONLY, QONLY)

        plsc.subcore_barrier()

        @pl.when(cid == FASTCID)
        def _():
            pltpu.sync_copy(shared.at[pl.ds(sid * STRIPE, STRIPE)],
                            out_hbm.at[pl.ds(sid * STRIPE, STRIPE)])

        @pl.when((sid == 15) & (cid == FASTCID))
        def _():
            pltpu.sync_copy(shared.at[pl.ds(16 * STRIPE, LASTROWS - STRIPE)],
                            out_hbm.at[pl.ds(16 * STRIPE,
                                             LASTROWS - STRIPE)])

    return k


def _sc_scatter(x, sd3, zeros):
    return _sc_scatter_fn()(x, sd3, zeros)


@functools.lru_cache(maxsize=None)
def _sc_degree_fn():
    mesh = plsc.VectorSubcoreMesh(core_axis_name="c", subcore_axis_name="s")

    @functools.partial(
        pl.kernel,
        mesh=mesh,
        out_type=jax.ShapeDtypeStruct((2, N, C), jnp.float32),
        scratch_types=[
            pltpu.VMEM((GROUP, CHUNK), jnp.int32),
            pltpu.VMEM((CHUNK, C), jnp.float32),
            pltpu.VMEM_SHARED((NACC, C), jnp.float32),
            pltpu.SemaphoreType.DMA,
        ],
    )
    def k(dst_hbm, ones_hbm, zeros_hbm, out_hbm, didx, ones_v, shared, sem):
        cid = lax.axis_index("c")
        sid = lax.axis_index("s")
        wid = sid * 2 + cid
        pltpu.sync_copy(ones_hbm, ones_v)
        pltpu.sync_copy(zeros_hbm.at[pl.ds(0, STRIPE)],
                        shared.at[pl.ds(sid * STRIPE, STRIPE)])

        @pl.when(sid == 15)
        def _():
            pltpu.sync_copy(zeros_hbm.at[pl.ds(0, LASTACC - STRIPE)],
                            shared.at[pl.ds(16 * STRIPE, LASTACC - STRIPE)])

        plsc.subcore_barrier()
        row0 = wid * WCHUNKS

        def group_body(i, _):
            g0 = row0 + i * GROUP
            pltpu.sync_copy(dst_hbm.at[pl.ds(g0, GROUP)], didx)
            cps = [pltpu.async_copy(ones_v, shared.at[didx.at[b]], sem,
                                    add=True)
                   for b in range(GROUP)]
            for cp in cps:
                cp.wait()
            return 0

        lax.fori_loop(0, NGROUPS, group_body, 0)
        plsc.subcore_barrier()
        pltpu.sync_copy(shared.at[pl.ds(sid * STRIPE, STRIPE)],
                        out_hbm.at[cid, pl.ds(sid * STRIPE, STRIPE)])

        @pl.when(sid == 15)
        def _():
            pltpu.sync_copy(shared.at[pl.ds(16 * STRIPE, LASTROWS - STRIPE)],
                            out_hbm.at[cid, pl.ds(16 * STRIPE,
                                                  LASTROWS - STRIPE)])

    return k


def _sc_degree(dst2d, ones80, zeros125):
    return _sc_degree_fn()(dst2d, ones80, zeros125)


_BLK = 1000
_NBLK = N // _BLK


def _enc_body(x_ref, w_ref, b_ref, o_ref):
    h = x_ref[...]
    for i in range(3):
        h = _leaky(_dot(h, w_ref[i]) + b_ref[i])
    o_ref[...] = h


def _encoder(x, enc_n_W, enc_n_b):
    return pl.pallas_call(
        _enc_body,
        grid=(_NBLK,),
        in_specs=[
            pl.BlockSpec((_BLK, C), lambda i: (i, 0)),
            pl.BlockSpec((3, C, C), lambda i: (0, 0, 0)),
            pl.BlockSpec((3, C), lambda i: (0, 0)),
        ],
        out_specs=pl.BlockSpec((_BLK, C), lambda i: (i, 0)),
        out_shape=jax.ShapeDtypeStruct((N, C), jnp.float32),
    )(x, enc_n_W, enc_n_b)


def _xw23_body(x_ref, w23_ref, o_ref):
    o_ref[...] = _dot(x_ref[...], w23_ref[...])


def _xw23(x, w23):
    return pl.pallas_call(
        _xw23_body,
        grid=(_NBLK,),
        in_specs=[
            pl.BlockSpec((_BLK, C), lambda i: (i, 0)),
            pl.BlockSpec((C, 2 * C), lambda i: (0, 0)),
        ],
        out_specs=pl.BlockSpec((_BLK, 2 * C), lambda i: (i, 0)),
        out_shape=jax.ShapeDtypeStruct((N, 2 * C), jnp.float32),
    )(x, w23)


def _combA_body(p_ref, u_ref, deg_ref, w1_ref, b1_ref,
                b3_ref, mw1_ref, mb1_ref, h1_ref, ssum_ref, ssq_ref):
    pid = pl.program_id(0)
    psum = p_ref[...]
    deg = deg_ref[0, :, 0:1] + deg_ref[1, :, 0:1]
    agg = _dot(psum, w1_ref[...]) + deg * b1_ref[...] - deg * u_ref[:, :C]
    h = agg + u_ref[:, C:] + b3_ref[...]
    h1 = _dot(h, mw1_ref[...]) + mb1_ref[...]
    h1_ref[...] = h1
    s = jnp.broadcast_to(jnp.sum(h1, axis=0, keepdims=True), (8, 2 * C))
    q = jnp.broadcast_to(jnp.sum(h1 * h1, axis=0, keepdims=True), (8, 2 * C))

    @pl.when(pid == 0)
    def _():
        ssum_ref[...] = s
        ssq_ref[...] = q

    @pl.when(pid > 0)
    def _():
        ssum_ref[...] += s
        ssq_ref[...] += q


def _combineA(P, U, degf, w1, b1, b3, mw1, mb1):
    return pl.pallas_call(
        _combA_body,
        grid=(_NBLK,),
        in_specs=[
            pl.BlockSpec((_BLK, C), lambda i: (i, 0)),
            pl.BlockSpec((_BLK, 2 * C), lambda i: (i, 0)),
            pl.BlockSpec((2, _BLK, C), lambda i: (0, i, 0)),
            pl.BlockSpec((C, C), lambda i: (0, 0)),
            pl.BlockSpec((1, C), lambda i: (0, 0)),
            pl.BlockSpec((1, C), lambda i: (0, 0)),
            pl.BlockSpec((C, 2 * C), lambda i: (0, 0)),
            pl.BlockSpec((1, 2 * C), lambda i: (0, 0)),
        ],
        out_specs=[
            pl.BlockSpec((_BLK, 2 * C), lambda i: (i, 0)),
            pl.BlockSpec((8, 2 * C), lambda i: (0, 0)),
            pl.BlockSpec((8, 2 * C), lambda i: (0, 0)),
        ],
        out_shape=[
            jax.ShapeDtypeStruct((N, 2 * C), jnp.float32),
            jax.ShapeDtypeStruct((8, 2 * C), jnp.float32),
            jax.ShapeDtypeStruct((8, 2 * C), jnp.float32),
        ],
    )(P, U, degf, w1, b1, b3, mw1, mb1)


def _make_combB_body(with_skip):
    def body(*refs):
        if with_skip:
            (h1_ref, ssum_ref, ssq_ref, g_ref, beta_ref, mw2_ref, mb2_ref,
             skip_ref, o_ref) = refs
        else:
            (h1_ref, ssum_ref, ssq_ref, g_ref, beta_ref, mw2_ref, mb2_ref,
             o_ref) = refs
        mean = ssum_ref[0:1, :] * (1.0 / N)
        var = ssq_ref[0:1, :] * (1.0 / N) - mean * mean
        hn = (h1_ref[...] - mean) * lax.rsqrt(var + 1e-5) * g_ref[...] \
            + beta_ref[...]
        hn = _leaky(hn)
        y = _dot(hn, mw2_ref[...]) + mb2_ref[...]
        if with_skip:
            y = y + skip_ref[...]
        o_ref[...] = y
    return body


def _combineB(h1, ssum, ssq, g, beta, mw2, mb2, skip):
    in_specs = [
        pl.BlockSpec((_BLK, 2 * C), lambda i: (i, 0)),
        pl.BlockSpec((8, 2 * C), lambda i: (0, 0)),
        pl.BlockSpec((8, 2 * C), lambda i: (0, 0)),
        pl.BlockSpec((1, 2 * C), lambda i: (0, 0)),
        pl.BlockSpec((1, 2 * C), lambda i: (0, 0)),
        pl.BlockSpec((2 * C, C), lambda i: (0, 0)),
        pl.BlockSpec((1, C), lambda i: (0, 0)),
    ]
    args = [h1, ssum, ssq, g, beta, mw2, mb2]
    if skip is not None:
        in_specs.append(pl.BlockSpec((_BLK, C), lambda i: (i, 0)))
        args.append(skip)
    return pl.pallas_call(
        _make_combB_body(skip is not None),
        grid=(_NBLK,),
        in_specs=in_specs,
        out_specs=pl.BlockSpec((_BLK, C), lambda i: (i, 0)),
        out_shape=jax.ShapeDtypeStruct((N, C), jnp.float32),
    )(*args)


def _pool_body(x_ref, b_ref, ow_ref, ob_ref, wf_ref, bf_ref, o_ref,
               sums, cnts):
    pid = pl.program_id(0)

    @pl.when(pid == 0)
    def _():
        sums[...] = jnp.zeros((G, C), jnp.float32)
        cnts[...] = jnp.zeros((G, C), jnp.float32)

    bb = b_ref[0]
    gi = lax.broadcasted_iota(jnp.int32, (G, _BLK), 0)
    m = (gi == jnp.broadcast_to(bb, (G, _BLK))).astype(jnp.float32)
    sums[...] += _dot(m, x_ref[...])
    cnts[...] += jnp.broadcast_to(
        jnp.sum(m, axis=1, keepdims=True), (G, C))

    @pl.when(pid == _NBLK - 1)
    def _():
        pooled = sums[...] / jnp.maximum(cnts[...], 1.0)
        o = _leaky(_dot(pooled, ow_ref[0]) + ob_ref[0:1, :])
        o = _leaky(_dot(o, ow_ref[1]) + ob_ref[1:2, :])
        o_ref[...] = _dot(o, wf_ref[...]) + bf_ref[...]


def _pool_head(x, batch3, out_W, out_b, wf_pad, bf_pad):
    return pl.pallas_call(
        _pool_body,
        grid=(_NBLK,),
        in_specs=[
            pl.BlockSpec((_BLK, C), lambda i: (i, 0)),
            pl.BlockSpec((1, 1, _BLK), lambda i: (i, 0, 0)),
            pl.BlockSpec((2, C, C), lambda i: (0, 0, 0)),
            pl.BlockSpec((2, C), lambda i: (0, 0)),
            pl.BlockSpec((C, C), lambda i: (0, 0)),
            pl.BlockSpec((1, C), lambda i: (0, 0)),
        ],
        out_specs=pl.BlockSpec((G, C), lambda i: (0, 0)),
        out_shape=jax.ShapeDtypeStruct((G, C), jnp.float32),
        scratch_shapes=[
            pltpu.VMEM((G, C), jnp.float32),
            pltpu.VMEM((G, C), jnp.float32),
        ],
    )(x, batch3, out_W, out_b, wf_pad, bf_pad)


def kernel(x, edge_attr, enc_n_W, enc_n_b, enc_e_W0, enc_e_b0, enc_e_W,
           enc_e_b, conv_W1, conv_b1, conv_W2, conv_W3, conv_b3,
           mlp_W1, mlp_b1, mlp_g, mlp_beta, mlp_W2, mlp_b2,
           out_W, out_b, out_Wf, out_bf, edge_index, batch, ptr):
    pad_src = jnp.zeros((EPAD,), jnp.int32)
    pad_dst = jnp.full((EPAD,), N, jnp.int32)
    src2d = jnp.concatenate([edge_index[0], pad_src]).reshape(
        NROWCHUNKS, CHUNK)
    dst2d = jnp.concatenate([edge_index[1], pad_dst]).reshape(
        NROWCHUNKS, CHUNK)
    sd3 = jnp.stack([src2d, dst2d], axis=1)
    zeros_x = jnp.zeros((STRIPE, C), jnp.float32)
    ones_e = jnp.ones((CHUNK, C), jnp.float32)

    xe = _encoder(x, enc_n_W, enc_n_b)
    degf = _sc_degree(dst2d, ones_e, zeros_x)

    xc = xe
    for i in range(3):
        U = _xw23(xc, jnp.concatenate([conv_W2[i], conv_W3[i]], axis=1))
        P = _sc_scatter(xc, sd3, zeros_x)
        h1, ssum, ssq = _combineA(
            P, U, degf,
            conv_W1[i], conv_b1[i].reshape(1, C),
            conv_b3[i].reshape(1, C), mlp_W1[i], mlp_b1[i].reshape(1, 2 * C))
        xc = _combineB(
            h1, ssum, ssq, mlp_g[i].reshape(1, 2 * C),
            mlp_beta[i].reshape(1, 2 * C), mlp_W2[i],
            mlp_b2[i].reshape(1, C), xe if i == 0 else None)

    batch3 = batch.reshape(_NBLK, 1, _BLK)
    wf_pad = jnp.concatenate(
        [out_Wf, jnp.zeros((C, C - 1), jnp.float32)], axis=1)
    bf_pad = jnp.broadcast_to(out_bf.reshape(1, 1), (1, C))
    o = _pool_head(xc, batch3, out_W, out_b, wf_pad, bf_pad)
    return o[:, :1]


# speedup vs baseline: 1.0940x; 1.0940x over previous
import functools

import jax
import jax.numpy as jnp
from jax import lax
from jax.experimental import pallas as pl
from jax.experimental.pallas import tpu as pltpu
from jax.experimental.pallas import tpu_sc as plsc

N = 10000
E = 320000
C = 128
G = 16
NW = 32
CHUNK = 128
NROWCHUNKS = 2560
EPAD = NROWCHUNKS * CHUNK - E
NACC = N + 8
GROUP = 8
WCHUNKS = NROWCHUNKS // NW
NGROUPS = WCHUNKS // GROUP
DEPTH = 2
STRIPE = 624
LASTROWS = N - 15 * STRIPE
LASTACC = NACC - 15 * STRIPE

_HIGH = jax.lax.Precision.HIGHEST
IB = 16
NB = WCHUNKS // IB


def _leaky(v):
    return jnp.where(v > 0, v, 0.01 * v)


def _dot(a, b):
    return jnp.dot(a, b, precision=_HIGH, preferred_element_type=jnp.float32)


@functools.lru_cache(maxsize=None)
def _sc_scatter_fn():
    mesh = plsc.VectorSubcoreMesh(core_axis_name="c", subcore_axis_name="s")

    @functools.partial(
        pl.kernel,
        mesh=mesh,
        out_type=jax.ShapeDtypeStruct((2, N, C), jnp.float32),
        scratch_types=[
            pltpu.VMEM((IB, 2, CHUNK), jnp.int32),
            pltpu.VMEM((IB, 2, CHUNK), jnp.int32),
            *[pltpu.VMEM((CHUNK, C), jnp.float32) for _ in range(DEPTH)],
            pltpu.VMEM_SHARED((NACC, C), jnp.float32),
            pltpu.SemaphoreType.DMA,
            pltpu.SemaphoreType.DMA,
            pltpu.SemaphoreType.DMA,
            pltpu.SemaphoreType.DMA,
        ],
    )
    def k(x_hbm, sd_hbm, zeros_hbm, out_hbm,
          ib0, ib1, r0, r1, shared, isem, gsem0, gsem1, ssem):
        rows = [r0, r1]
        gsems = [gsem0, gsem1]
        ibufs = [ib0, ib1]
        cid = lax.axis_index("c")
        sid = lax.axis_index("s")
        wid = sid * 2 + cid

        pltpu.sync_copy(zeros_hbm.at[pl.ds(0, STRIPE)],
                        shared.at[pl.ds(sid * STRIPE, STRIPE)])

        @pl.when(sid == 15)
        def _():
            pltpu.sync_copy(zeros_hbm.at[pl.ds(0, LASTACC - STRIPE)],
                            shared.at[pl.ds(16 * STRIPE,
                                            LASTACC - STRIPE)])

        plsc.subcore_barrier()

        def _ring(base, nch):
            nbk = nch // IB
            icps = [None] * nbk
            icps[0] = pltpu.async_copy(sd_hbm.at[pl.ds(base, IB)], ib0,
                                       isem)
            gcps = [None] * nch
            scps = [None] * nch

            def _scat(j):
                gcps[j].wait()
                jb = ibufs[(j // IB) % 2]
                scps[j] = pltpu.async_copy(
                    rows[j % DEPTH], shared.at[jb.at[j % IB, 1]], ssem,
                    add=True)

            for c in range(nch):
                blk, off = divmod(c, IB)
                if off == 0:
                    icps[blk].wait()
                if off == 2 and blk + 1 < nbk:
                    icps[blk + 1] = pltpu.async_copy(
                        sd_hbm.at[pl.ds(base + (blk + 1) * IB, IB)],
                        ibufs[(blk + 1) % 2], isem)
                if c >= DEPTH:
                    scps[c - DEPTH].wait()
                gcps[c] = pltpu.async_copy(
                    x_hbm.at[ibufs[blk % 2].at[off, 0]], rows[c % DEPTH],
                    gsems[c % 2])
                if c >= 1:
                    _scat(c - 1)
            _scat(nch - 1)
            scps[nch - 2].wait()
            scps[nch - 1].wait()

        _ring(wid * WCHUNKS, WCHUNKS)

        plsc.subcore_barrier()

        pltpu.sync_copy(shared.at[pl.ds(sid * STRIPE, STRIPE)],
                        out_hbm.at[cid, pl.ds(sid * STRIPE, STRIPE)])

        @pl.when(sid == 15)
        def _():
            pltpu.sync_copy(shared.at[pl.ds(16 * STRIPE, LASTROWS - STRIPE)],
                            out_hbm.at[cid, pl.ds(16 * STRIPE,
                                                  LASTROWS - STRIPE)])

    return k


def _sc_scatter(x, sd3, zeros):
    return _sc_scatter_fn()(x, sd3, zeros)


@functools.lru_cache(maxsize=None)
def _sc_degree_fn():
    mesh = plsc.VectorSubcoreMesh(core_axis_name="c", subcore_axis_name="s")

    @functools.partial(
        pl.kernel,
        mesh=mesh,
        out_type=jax.ShapeDtypeStruct((2, N, C), jnp.float32),
        scratch_types=[
            pltpu.VMEM((GROUP, CHUNK), jnp.int32),
            pltpu.VMEM((CHUNK, C), jnp.float32),
            pltpu.VMEM_SHARED((NACC, C), jnp.float32),
            pltpu.SemaphoreType.DMA,
        ],
    )
    def k(dst_hbm, ones_hbm, zeros_hbm, out_hbm, didx, ones_v, shared, sem):
        cid = lax.axis_index("c")
        sid = lax.axis_index("s")
        wid = sid * 2 + cid
        pltpu.sync_copy(ones_hbm, ones_v)
        pltpu.sync_copy(zeros_hbm.at[pl.ds(0, STRIPE)],
                        shared.at[pl.ds(sid * STRIPE, STRIPE)])

        @pl.when(sid == 15)
        def _():
            pltpu.sync_copy(zeros_hbm.at[pl.ds(0, LASTACC - STRIPE)],
                            shared.at[pl.ds(16 * STRIPE, LASTACC - STRIPE)])

        plsc.subcore_barrier()
        row0 = wid * WCHUNKS

        def group_body(i, _):
            g0 = row0 + i * GROUP
            pltpu.sync_copy(dst_hbm.at[pl.ds(g0, GROUP)], didx)
            cps = [pltpu.async_copy(ones_v, shared.at[didx.at[b]], sem,
                                    add=True)
                   for b in range(GROUP)]
            for cp in cps:
                cp.wait()
            return 0

        lax.fori_loop(0, NGROUPS, group_body, 0)
        plsc.subcore_barrier()
        pltpu.sync_copy(shared.at[pl.ds(sid * STRIPE, STRIPE)],
                        out_hbm.at[cid, pl.ds(sid * STRIPE, STRIPE)])

        @pl.when(sid == 15)
        def _():
            pltpu.sync_copy(shared.at[pl.ds(16 * STRIPE, LASTROWS - STRIPE)],
                            out_hbm.at[cid, pl.ds(16 * STRIPE,
                                                  LASTROWS - STRIPE)])

    return k


def _sc_degree(dst2d, ones80, zeros125):
    return _sc_degree_fn()(dst2d, ones80, zeros125)


_BLK = 1000
_NBLK = N // _BLK


def _enc_body(x_ref, w_ref, b_ref, o_ref):
    h = x_ref[...]
    for i in range(3):
        h = _leaky(_dot(h, w_ref[i]) + b_ref[i])
    o_ref[...] = h


def _encoder(x, enc_n_W, enc_n_b):
    return pl.pallas_call(
        _enc_body,
        grid=(_NBLK,),
        in_specs=[
            pl.BlockSpec((_BLK, C), lambda i: (i, 0)),
            pl.BlockSpec((3, C, C), lambda i: (0, 0, 0)),
            pl.BlockSpec((3, C), lambda i: (0, 0)),
        ],
        out_specs=pl.BlockSpec((_BLK, C), lambda i: (i, 0)),
        out_shape=jax.ShapeDtypeStruct((N, C), jnp.float32),
    )(x, enc_n_W, enc_n_b)


def _xw23_body(x_ref, w23_ref, o_ref):
    o_ref[...] = _dot(x_ref[...], w23_ref[...])


def _xw23(x, w23):
    return pl.pallas_call(
        _xw23_body,
        grid=(_NBLK,),
        in_specs=[
            pl.BlockSpec((_BLK, C), lambda i: (i, 0)),
            pl.BlockSpec((C, 2 * C), lambda i: (0, 0)),
        ],
        out_specs=pl.BlockSpec((_BLK, 2 * C), lambda i: (i, 0)),
        out_shape=jax.ShapeDtypeStruct((N, 2 * C), jnp.float32),
    )(x, w23)


def _combA_body(p_ref, u_ref, deg_ref, w1_ref, b1_ref,
                b3_ref, mw1_ref, mb1_ref, h1_ref, ssum_ref, ssq_ref):
    pid = pl.program_id(0)
    psum = p_ref[0] + p_ref[1]
    deg = deg_ref[0, :, 0:1] + deg_ref[1, :, 0:1]
    agg = _dot(psum, w1_ref[...]) + deg * b1_ref[...] - deg * u_ref[:, :C]
    h = agg + u_ref[:, C:] + b3_ref[...]
    h1 = _dot(h, mw1_ref[...]) + mb1_ref[...]
    h1_ref[...] = h1
    s = jnp.broadcast_to(jnp.sum(h1, axis=0, keepdims=True), (8, 2 * C))
    q = jnp.broadcast_to(jnp.sum(h1 * h1, axis=0, keepdims=True), (8, 2 * C))

    @pl.when(pid == 0)
    def _():
        ssum_ref[...] = s
        ssq_ref[...] = q

    @pl.when(pid > 0)
    def _():
        ssum_ref[...] += s
        ssq_ref[...] += q


def _combineA(P, U, degf, w1, b1, b3, mw1, mb1):
    return pl.pallas_call(
        _combA_body,
        grid=(_NBLK,),
        in_specs=[
            pl.BlockSpec((2, _BLK, C), lambda i: (0, i, 0)),
            pl.BlockSpec((_BLK, 2 * C), lambda i: (i, 0)),
            pl.BlockSpec((2, _BLK, C), lambda i: (0, i, 0)),
            pl.BlockSpec((C, C), lambda i: (0, 0)),
            pl.BlockSpec((1, C), lambda i: (0, 0)),
            pl.BlockSpec((1, C), lambda i: (0, 0)),
            pl.BlockSpec((C, 2 * C), lambda i: (0, 0)),
            pl.BlockSpec((1, 2 * C), lambda i: (0, 0)),
        ],
        out_specs=[
            pl.BlockSpec((_BLK, 2 * C), lambda i: (i, 0)),
            pl.BlockSpec((8, 2 * C), lambda i: (0, 0)),
            pl.BlockSpec((8, 2 * C), lambda i: (0, 0)),
        ],
        out_shape=[
            jax.ShapeDtypeStruct((N, 2 * C), jnp.float32),
            jax.ShapeDtypeStruct((8, 2 * C), jnp.float32),
            jax.ShapeDtypeStruct((8, 2 * C), jnp.float32),
        ],
    )(P, U, degf, w1, b1, b3, mw1, mb1)


def _make_combB_body(with_skip):
    def body(*refs):
        if with_skip:
            (h1_ref, ssum_ref, ssq_ref, g_ref, beta_ref, mw2_ref, mb2_ref,
             skip_ref, o_ref) = refs
        else:
            (h1_ref, ssum_ref, ssq_ref, g_ref, beta_ref, mw2_ref, mb2_ref,
             o_ref) = refs
        mean = ssum_ref[0:1, :] * (1.0 / N)
        var = ssq_ref[0:1, :] * (1.0 / N) - mean * mean
        hn = (h1_ref[...] - mean) * lax.rsqrt(var + 1e-5) * g_ref[...] \
            + beta_ref[...]
        hn = _leaky(hn)
        y = _dot(hn, mw2_ref[...]) + mb2_ref[...]
        if with_skip:
            y = y + skip_ref[...]
        o_ref[...] = y
    return body


def _combineB(h1, ssum, ssq, g, beta, mw2, mb2, skip):
    in_specs = [
        pl.BlockSpec((_BLK, 2 * C), lambda i: (i, 0)),
        pl.BlockSpec((8, 2 * C), lambda i: (0, 0)),
        pl.BlockSpec((8, 2 * C), lambda i: (0, 0)),
        pl.BlockSpec((1, 2 * C), lambda i: (0, 0)),
        pl.BlockSpec((1, 2 * C), lambda i: (0, 0)),
        pl.BlockSpec((2 * C, C), lambda i: (0, 0)),
        pl.BlockSpec((1, C), lambda i: (0, 0)),
    ]
    args = [h1, ssum, ssq, g, beta, mw2, mb2]
    if skip is not None:
        in_specs.append(pl.BlockSpec((_BLK, C), lambda i: (i, 0)))
        args.append(skip)
    return pl.pallas_call(
        _make_combB_body(skip is not None),
        grid=(_NBLK,),
        in_specs=in_specs,
        out_specs=pl.BlockSpec((_BLK, C), lambda i: (i, 0)),
        out_shape=jax.ShapeDtypeStruct((N, C), jnp.float32),
    )(*args)


def _pool_body(x_ref, b_ref, ow_ref, ob_ref, wf_ref, bf_ref, o_ref,
               sums, cnts):
    pid = pl.program_id(0)

    @pl.when(pid == 0)
    def _():
        sums[...] = jnp.zeros((G, C), jnp.float32)
        cnts[...] = jnp.zeros((G, C), jnp.float32)

    bb = b_ref[0]
    gi = lax.broadcasted_iota(jnp.int32, (G, _BLK), 0)
    m = (gi == jnp.broadcast_to(bb, (G, _BLK))).astype(jnp.float32)
    sums[...] += _dot(m, x_ref[...])
    cnts[...] += jnp.broadcast_to(
        jnp.sum(m, axis=1, keepdims=True), (G, C))

    @pl.when(pid == _NBLK - 1)
    def _():
        pooled = sums[...] / jnp.maximum(cnts[...], 1.0)
        o = _leaky(_dot(pooled, ow_ref[0]) + ob_ref[0:1, :])
        o = _leaky(_dot(o, ow_ref[1]) + ob_ref[1:2, :])
        o_ref[...] = _dot(o, wf_ref[...]) + bf_ref[...]


def _pool_head(x, batch3, out_W, out_b, wf_pad, bf_pad):
    return pl.pallas_call(
        _pool_body,
        grid=(_NBLK,),
        in_specs=[
            pl.BlockSpec((_BLK, C), lambda i: (i, 0)),
            pl.BlockSpec((1, 1, _BLK), lambda i: (i, 0, 0)),
            pl.BlockSpec((2, C, C), lambda i: (0, 0, 0)),
            pl.BlockSpec((2, C), lambda i: (0, 0)),
            pl.BlockSpec((C, C), lambda i: (0, 0)),
            pl.BlockSpec((1, C), lambda i: (0, 0)),
        ],
        out_specs=pl.BlockSpec((G, C), lambda i: (0, 0)),
        out_shape=jax.ShapeDtypeStruct((G, C), jnp.float32),
        scratch_shapes=[
            pltpu.VMEM((G, C), jnp.float32),
            pltpu.VMEM((G, C), jnp.float32),
        ],
    )(x, batch3, out_W, out_b, wf_pad, bf_pad)


def kernel(x, edge_attr, enc_n_W, enc_n_b, enc_e_W0, enc_e_b0, enc_e_W,
           enc_e_b, conv_W1, conv_b1, conv_W2, conv_W3, conv_b3,
           mlp_W1, mlp_b1, mlp_g, mlp_beta, mlp_W2, mlp_b2,
           out_W, out_b, out_Wf, out_bf, edge_index, batch, ptr):
    pad_src = jnp.zeros((EPAD,), jnp.int32)
    pad_dst = jnp.full((EPAD,), N, jnp.int32)
    src2d = jnp.concatenate([edge_index[0], pad_src]).reshape(
        NROWCHUNKS, CHUNK)
    dst2d = jnp.concatenate([edge_index[1], pad_dst]).reshape(
        NROWCHUNKS, CHUNK)
    sd3 = jnp.stack([src2d, dst2d], axis=1)
    zeros_x = jnp.zeros((STRIPE, C), jnp.float32)
    ones_e = jnp.ones((CHUNK, C), jnp.float32)

    xe = _encoder(x, enc_n_W, enc_n_b)
    degf = _sc_degree(dst2d, ones_e, zeros_x)

    xc = xe
    for i in range(3):
        U = _xw23(xc, jnp.concatenate([conv_W2[i], conv_W3[i]], axis=1))
        P = _sc_scatter(xc, sd3, zeros_x)
        h1, ssum, ssq = _combineA(
            P, U, degf,
            conv_W1[i], conv_b1[i].reshape(1, C),
            conv_b3[i].reshape(1, C), mlp_W1[i], mlp_b1[i].reshape(1, 2 * C))
        xc = _combineB(
            h1, ssum, ssq, mlp_g[i].reshape(1, 2 * C),
            mlp_beta[i].reshape(1, 2 * C), mlp_W2[i],
            mlp_b2[i].reshape(1, C), xe if i == 0 else None)

    batch3 = batch.reshape(_NBLK, 1, _BLK)
    wf_pad = jnp.concatenate(
        [out_Wf, jnp.zeros((C, C - 1), jnp.float32)], axis=1)
    bf_pad = jnp.broadcast_to(out_bf.reshape(1, 1), (1, C))
    o = _pool_head(xc, batch3, out_W, out_b, wf_pad, bf_pad)
    return o[:, :1]


# speedup vs baseline: 1.1141x; 1.0183x over previous
import functools

import jax
import jax.numpy as jnp
from jax import lax
from jax.experimental import pallas as pl
from jax.experimental.pallas import tpu as pltpu
from jax.experimental.pallas import tpu_sc as plsc

N = 10000
E = 320000
C = 128
G = 16
NW = 32
CHUNK = 128
NROWCHUNKS = 2560
EPAD = NROWCHUNKS * CHUNK - E
NACC = N + 8
GROUP = 8
WCHUNKS = NROWCHUNKS // NW
NGROUPS = WCHUNKS // GROUP
DEPTH = 2
STRIPE = 624
LASTROWS = N - 15 * STRIPE
LASTACC = NACC - 15 * STRIPE

_HIGH = jax.lax.Precision.HIGHEST
IB = 16
NB = WCHUNKS // IB


def _leaky(v):
    return jnp.where(v > 0, v, 0.01 * v)


def _dot(a, b):
    return jnp.dot(a, b, precision=_HIGH, preferred_element_type=jnp.float32)


def _dotd(a, b):
    return jnp.dot(a, b, preferred_element_type=jnp.float32)


@functools.lru_cache(maxsize=None)
def _sc_scatter_fn():
    mesh = plsc.VectorSubcoreMesh(core_axis_name="c", subcore_axis_name="s")

    @functools.partial(
        pl.kernel,
        mesh=mesh,
        out_type=jax.ShapeDtypeStruct((2, N, C), jnp.float32),
        scratch_types=[
            pltpu.VMEM((IB, 2, CHUNK), jnp.int32),
            pltpu.VMEM((IB, 2, CHUNK), jnp.int32),
            *[pltpu.VMEM((CHUNK, C), jnp.float32) for _ in range(DEPTH)],
            pltpu.VMEM_SHARED((NACC, C), jnp.float32),
            pltpu.SemaphoreType.DMA,
            pltpu.SemaphoreType.DMA,
            pltpu.SemaphoreType.DMA,
            pltpu.SemaphoreType.DMA,
        ],
    )
    def k(x_hbm, sd_hbm, zeros_hbm, out_hbm,
          ib0, ib1, r0, r1, shared, isem, gsem0, gsem1, ssem):
        rows = [r0, r1]
        gsems = [gsem0, gsem1]
        ibufs = [ib0, ib1]
        cid = lax.axis_index("c")
        sid = lax.axis_index("s")
        wid = sid * 2 + cid

        pltpu.sync_copy(zeros_hbm.at[pl.ds(0, STRIPE)],
                        shared.at[pl.ds(sid * STRIPE, STRIPE)])

        @pl.when(sid == 15)
        def _():
            pltpu.sync_copy(zeros_hbm.at[pl.ds(0, LASTACC - STRIPE)],
                            shared.at[pl.ds(16 * STRIPE,
                                            LASTACC - STRIPE)])

        plsc.subcore_barrier()

        def _ring(base, nch):
            nbk = nch // IB
            icps = [None] * nbk
            icps[0] = pltpu.async_copy(sd_hbm.at[pl.ds(base, IB)], ib0,
                                       isem)
            gcps = [None] * nch
            scps = [None] * nch

            def _scat(j):
                gcps[j].wait()
                jb = ibufs[(j // IB) % 2]
                scps[j] = pltpu.async_copy(
                    rows[j % DEPTH], shared.at[jb.at[j % IB, 1]], ssem,
                    add=True)

            for c in range(nch):
                blk, off = divmod(c, IB)
                if off == 0:
                    icps[blk].wait()
                if off == 2 and blk + 1 < nbk:
                    icps[blk + 1] = pltpu.async_copy(
                        sd_hbm.at[pl.ds(base + (blk + 1) * IB, IB)],
                        ibufs[(blk + 1) % 2], isem)
                if c >= DEPTH:
                    scps[c - DEPTH].wait()
                gcps[c] = pltpu.async_copy(
                    x_hbm.at[ibufs[blk % 2].at[off, 0]], rows[c % DEPTH],
                    gsems[c % 2])
                if c >= 1:
                    _scat(c - 1)
            _scat(nch - 1)
            scps[nch - 2].wait()
            scps[nch - 1].wait()

        _ring(wid * WCHUNKS, WCHUNKS)

        plsc.subcore_barrier()

        pltpu.sync_copy(shared.at[pl.ds(sid * STRIPE, STRIPE)],
                        out_hbm.at[cid, pl.ds(sid * STRIPE, STRIPE)])

        @pl.when(sid == 15)
        def _():
            pltpu.sync_copy(shared.at[pl.ds(16 * STRIPE, LASTROWS - STRIPE)],
                            out_hbm.at[cid, pl.ds(16 * STRIPE,
                                                  LASTROWS - STRIPE)])

    return k


def _sc_scatter(x, sd3, zeros):
    return _sc_scatter_fn()(x, sd3, zeros)


@functools.lru_cache(maxsize=None)
def _sc_degree_fn():
    mesh = plsc.VectorSubcoreMesh(core_axis_name="c", subcore_axis_name="s")

    @functools.partial(
        pl.kernel,
        mesh=mesh,
        out_type=jax.ShapeDtypeStruct((2, N, C), jnp.float32),
        scratch_types=[
            pltpu.VMEM((GROUP, CHUNK), jnp.int32),
            pltpu.VMEM((CHUNK, C), jnp.float32),
            pltpu.VMEM_SHARED((NACC, C), jnp.float32),
            pltpu.SemaphoreType.DMA,
        ],
    )
    def k(dst_hbm, ones_hbm, zeros_hbm, out_hbm, didx, ones_v, shared, sem):
        cid = lax.axis_index("c")
        sid = lax.axis_index("s")
        wid = sid * 2 + cid
        pltpu.sync_copy(ones_hbm, ones_v)
        pltpu.sync_copy(zeros_hbm.at[pl.ds(0, STRIPE)],
                        shared.at[pl.ds(sid * STRIPE, STRIPE)])

        @pl.when(sid == 15)
        def _():
            pltpu.sync_copy(zeros_hbm.at[pl.ds(0, LASTACC - STRIPE)],
                            shared.at[pl.ds(16 * STRIPE, LASTACC - STRIPE)])

        plsc.subcore_barrier()
        row0 = wid * WCHUNKS

        def group_body(i, _):
            g0 = row0 + i * GROUP
            pltpu.sync_copy(dst_hbm.at[pl.ds(g0, GROUP)], didx)
            cps = [pltpu.async_copy(ones_v, shared.at[didx.at[b]], sem,
                                    add=True)
                   for b in range(GROUP)]
            for cp in cps:
                cp.wait()
            return 0

        lax.fori_loop(0, NGROUPS, group_body, 0)
        plsc.subcore_barrier()
        pltpu.sync_copy(shared.at[pl.ds(sid * STRIPE, STRIPE)],
                        out_hbm.at[cid, pl.ds(sid * STRIPE, STRIPE)])

        @pl.when(sid == 15)
        def _():
            pltpu.sync_copy(shared.at[pl.ds(16 * STRIPE, LASTROWS - STRIPE)],
                            out_hbm.at[cid, pl.ds(16 * STRIPE,
                                                  LASTROWS - STRIPE)])

    return k


def _sc_degree(dst2d, ones80, zeros125):
    return _sc_degree_fn()(dst2d, ones80, zeros125)


_BLK = 1000
_NBLK = N // _BLK


def _enc_body(x_ref, w_ref, b_ref, o_ref):
    h = x_ref[...]
    for i in range(3):
        h = _leaky(_dotd(h, w_ref[i]) + b_ref[i])
    o_ref[...] = h


def _encoder(x, enc_n_W, enc_n_b):
    return pl.pallas_call(
        _enc_body,
        grid=(_NBLK,),
        in_specs=[
            pl.BlockSpec((_BLK, C), lambda i: (i, 0)),
            pl.BlockSpec((3, C, C), lambda i: (0, 0, 0)),
            pl.BlockSpec((3, C), lambda i: (0, 0)),
        ],
        out_specs=pl.BlockSpec((_BLK, C), lambda i: (i, 0)),
        out_shape=jax.ShapeDtypeStruct((N, C), jnp.float32),
    )(x, enc_n_W, enc_n_b)


def _pre_body(x_ref, w123_ref, b1_ref, a_ref, u_ref):
    xw = _dotd(x_ref[...], w123_ref[...])
    a_ref[...] = xw[:, :C] + b1_ref[...]
    u_ref[...] = xw[:, C:]


def _pre(x, w123, b1):
    return pl.pallas_call(
        _pre_body,
        grid=(_NBLK,),
        in_specs=[
            pl.BlockSpec((_BLK, C), lambda i: (i, 0)),
            pl.BlockSpec((C, 3 * C), lambda i: (0, 0)),
            pl.BlockSpec((1, C), lambda i: (0, 0)),
        ],
        out_specs=[
            pl.BlockSpec((_BLK, C), lambda i: (i, 0)),
            pl.BlockSpec((_BLK, 2 * C), lambda i: (i, 0)),
        ],
        out_shape=[
            jax.ShapeDtypeStruct((N, C), jnp.float32),
            jax.ShapeDtypeStruct((N, 2 * C), jnp.float32),
        ],
    )(x, w123, b1)


def _combA_body(p_ref, u_ref, deg_ref,
                b3_ref, mw1_ref, mb1_ref, h1_ref, ssum_ref, ssq_ref):
    pid = pl.program_id(0)
    psum = p_ref[0] + p_ref[1]
    deg = deg_ref[0, :, 0:1] + deg_ref[1, :, 0:1]
    agg = psum - deg * u_ref[:, :C]
    h = agg + u_ref[:, C:] + b3_ref[...]
    h1 = _dotd(h, mw1_ref[...]) + mb1_ref[...]
    h1_ref[...] = h1
    s = jnp.broadcast_to(jnp.sum(h1, axis=0, keepdims=True), (8, 2 * C))
    q = jnp.broadcast_to(jnp.sum(h1 * h1, axis=0, keepdims=True), (8, 2 * C))

    @pl.when(pid == 0)
    def _():
        ssum_ref[...] = s
        ssq_ref[...] = q

    @pl.when(pid > 0)
    def _():
        ssum_ref[...] += s
        ssq_ref[...] += q


def _combineA(P, U, degf, b3, mw1, mb1):
    return pl.pallas_call(
        _combA_body,
        grid=(_NBLK,),
        in_specs=[
            pl.BlockSpec((2, _BLK, C), lambda i: (0, i, 0)),
            pl.BlockSpec((_BLK, 2 * C), lambda i: (i, 0)),
            pl.BlockSpec((2, _BLK, C), lambda i: (0, i, 0)),
            pl.BlockSpec((1, C), lambda i: (0, 0)),
            pl.BlockSpec((C, 2 * C), lambda i: (0, 0)),
            pl.BlockSpec((1, 2 * C), lambda i: (0, 0)),
        ],
        out_specs=[
            pl.BlockSpec((_BLK, 2 * C), lambda i: (i, 0)),
            pl.BlockSpec((8, 2 * C), lambda i: (0, 0)),
            pl.BlockSpec((8, 2 * C), lambda i: (0, 0)),
        ],
        out_shape=[
            jax.ShapeDtypeStruct((N, 2 * C), jnp.float32),
            jax.ShapeDtypeStruct((8, 2 * C), jnp.float32),
            jax.ShapeDtypeStruct((8, 2 * C), jnp.float32),
        ],
    )(P, U, degf, b3, mw1, mb1)


def _make_combB_body(with_skip):
    def body(*refs):
        if with_skip:
            (h1_ref, ssum_ref, ssq_ref, g_ref, beta_ref, mw2_ref, mb2_ref,
             skip_ref, o_ref) = refs
        else:
            (h1_ref, ssum_ref, ssq_ref, g_ref, beta_ref, mw2_ref, mb2_ref,
             o_ref) = refs
        mean = ssum_ref[0:1, :] * (1.0 / N)
        var = ssq_ref[0:1, :] * (1.0 / N) - mean * mean
        hn = (h1_ref[...] - mean) * lax.rsqrt(var + 1e-5) * g_ref[...] \
            + beta_ref[...]
        hn = _leaky(hn)
        y = _dotd(hn, mw2_ref[...]) + mb2_ref[...]
        if with_skip:
            y = y + skip_ref[...]
        o_ref[...] = y
    return body


def _combineB(h1, ssum, ssq, g, beta, mw2, mb2, skip):
    in_specs = [
        pl.BlockSpec((_BLK, 2 * C), lambda i: (i, 0)),
        pl.BlockSpec((8, 2 * C), lambda i: (0, 0)),
        pl.BlockSpec((8, 2 * C), lambda i: (0, 0)),
        pl.BlockSpec((1, 2 * C), lambda i: (0, 0)),
        pl.BlockSpec((1, 2 * C), lambda i: (0, 0)),
        pl.BlockSpec((2 * C, C), lambda i: (0, 0)),
        pl.BlockSpec((1, C), lambda i: (0, 0)),
    ]
    args = [h1, ssum, ssq, g, beta, mw2, mb2]
    if skip is not None:
        in_specs.append(pl.BlockSpec((_BLK, C), lambda i: (i, 0)))
        args.append(skip)
    return pl.pallas_call(
        _make_combB_body(skip is not None),
        grid=(_NBLK,),
        in_specs=in_specs,
        out_specs=pl.BlockSpec((_BLK, C), lambda i: (i, 0)),
        out_shape=jax.ShapeDtypeStruct((N, C), jnp.float32),
    )(*args)


def _pool_body(x_ref, b_ref, ow_ref, ob_ref, wf_ref, bf_ref, o_ref,
               sums, cnts):
    pid = pl.program_id(0)

    @pl.when(pid == 0)
    def _():
        sums[...] = jnp.zeros((G, C), jnp.float32)
        cnts[...] = jnp.zeros((G, C), jnp.float32)

    bb = b_ref[0]
    gi = lax.broadcasted_iota(jnp.int32, (G, _BLK), 0)
    m = (gi == jnp.broadcast_to(bb, (G, _BLK))).astype(jnp.float32)
    sums[...] += _dot(m, x_ref[...])
    cnts[...] += jnp.broadcast_to(
        jnp.sum(m, axis=1, keepdims=True), (G, C))

    @pl.when(pid == _NBLK - 1)
    def _():
        pooled = sums[...] / jnp.maximum(cnts[...], 1.0)
        o = _leaky(_dotd(pooled, ow_ref[0]) + ob_ref[0:1, :])
        o = _leaky(_dotd(o, ow_ref[1]) + ob_ref[1:2, :])
        o_ref[...] = _dotd(o, wf_ref[...]) + bf_ref[...]


def _pool_head(x, batch3, out_W, out_b, wf_pad, bf_pad):
    return pl.pallas_call(
        _pool_body,
        grid=(_NBLK,),
        in_specs=[
            pl.BlockSpec((_BLK, C), lambda i: (i, 0)),
            pl.BlockSpec((1, 1, _BLK), lambda i: (i, 0, 0)),
            pl.BlockSpec((2, C, C), lambda i: (0, 0, 0)),
            pl.BlockSpec((2, C), lambda i: (0, 0)),
            pl.BlockSpec((C, C), lambda i: (0, 0)),
            pl.BlockSpec((1, C), lambda i: (0, 0)),
        ],
        out_specs=pl.BlockSpec((G, C), lambda i: (0, 0)),
        out_shape=jax.ShapeDtypeStruct((G, C), jnp.float32),
        scratch_shapes=[
            pltpu.VMEM((G, C), jnp.float32),
            pltpu.VMEM((G, C), jnp.float32),
        ],
    )(x, batch3, out_W, out_b, wf_pad, bf_pad)


def kernel(x, edge_attr, enc_n_W, enc_n_b, enc_e_W0, enc_e_b0, enc_e_W,
           enc_e_b, conv_W1, conv_b1, conv_W2, conv_W3, conv_b3,
           mlp_W1, mlp_b1, mlp_g, mlp_beta, mlp_W2, mlp_b2,
           out_W, out_b, out_Wf, out_bf, edge_index, batch, ptr):
    pad_src = jnp.zeros((EPAD,), jnp.int32)
    pad_dst = jnp.full((EPAD,), N, jnp.int32)
    src2d = jnp.concatenate([edge_index[0], pad_src]).reshape(
        NROWCHUNKS, CHUNK)
    dst2d = jnp.concatenate([edge_index[1], pad_dst]).reshape(
        NROWCHUNKS, CHUNK)
    sd3 = jnp.stack([src2d, dst2d], axis=1)
    zeros_x = jnp.zeros((STRIPE, C), jnp.float32)
    ones_e = jnp.ones((CHUNK, C), jnp.float32)

    xe = _encoder(x, enc_n_W, enc_n_b)
    degf = _sc_degree(dst2d, ones_e, zeros_x)

    xc = xe
    for i in range(3):
        w123 = jnp.concatenate([conv_W1[i], conv_W2[i], conv_W3[i]], axis=1)
        A, U = _pre(xc, w123, conv_b1[i].reshape(1, C))
        P = _sc_scatter(A, sd3, zeros_x)
        h1, ssum, ssq = _combineA(
            P, U, degf,
            conv_b3[i].reshape(1, C), mlp_W1[i], mlp_b1[i].reshape(1, 2 * C))
        xc = _combineB(
            h1, ssum, ssq, mlp_g[i].reshape(1, 2 * C),
            mlp_beta[i].reshape(1, 2 * C), mlp_W2[i],
            mlp_b2[i].reshape(1, C), xe if i == 0 else None)

    batch3 = batch.reshape(_NBLK, 1, _BLK)
    wf_pad = jnp.concatenate(
        [out_Wf, jnp.zeros((C, C - 1), jnp.float32)], axis=1)
    bf_pad = jnp.broadcast_to(out_bf.reshape(1, 1), (1, C))
    o = _pool_head(xc, batch3, out_W, out_b, wf_pad, bf_pad)
    return o[:, :1]
